# Initial kernel scaffold; baseline (speedup 1.0000x reference)
import jax
import jax.numpy as jnp
from jax.experimental import pallas as pl


def kernel(sino, alpha, apod, k0, valid):
    raise NotImplementedError("write your pallas kernel here")


# speedup vs baseline: 621.8671x; 621.8671x over previous
import functools

import jax
import jax.numpy as jnp
from jax import lax
from jax.experimental import pallas as pl
from jax.experimental.pallas import tpu as pltpu
from jax.experimental.pallas import tpu_sc as plsc

B = 4
N_DET = 128
N_T = 2048
NY = 256
NX = 256
P = NY * NX
NC = 2
NS = 16
DETS_PER_TILE = N_DET // NS
P_HALF = P // NC
CHUNK = 1024
N_CHUNKS = P_HALF // CHUNK
PVECS = CHUNK // 16


def _normalize_body(sino_ref, out_ref):
    x = sino_ref[...]
    eps = jnp.finfo(jnp.float32).eps
    m = jnp.mean(x, axis=(1, 2), keepdims=True)
    c = x - m
    var = jnp.mean(c * c, axis=(1, 2), keepdims=True)
    s = c / jnp.sqrt(var + eps)
    out_ref[...] = jnp.transpose(s, (1, 0, 2))


def _lut_body(alpha_ref, valid_ref, k0_ref, apod_ref, w0_ref, w1_ref, k0t_ref):
    a = alpha_ref[...]
    v = valid_ref[...].astype(jnp.float32)
    ap = apod_ref[...]
    norm = jnp.maximum(jnp.sum(ap), jnp.finfo(jnp.float32).tiny)
    base = v * (ap / norm)
    w1 = base * a
    w0 = base - w1
    w0_ref[...] = w0.T
    w1_ref[...] = w1.T
    k0t_ref[...] = k0_ref[...].T


def _sc_body(s_hbm, k0_hbm, w0_hbm, w1_hbm, part_hbm,
             s_loc, k_loc, w0_loc, w1_loc, acc_loc):
    cid = lax.axis_index("c")
    sid = lax.axis_index("s")
    det0 = sid * DETS_PER_TILE

    pltpu.sync_copy(s_hbm.at[pl.ds(det0 * B, DETS_PER_TILE * B), :], s_loc)

    def chunk_body(c, _):
        col0 = cid * P_HALF + c * CHUNK
        pltpu.sync_copy(k0_hbm.at[pl.ds(det0, DETS_PER_TILE), pl.ds(col0, CHUNK)],
                        k_loc)
        pltpu.sync_copy(w0_hbm.at[pl.ds(det0, DETS_PER_TILE), pl.ds(col0, CHUNK)],
                        w0_loc)
        pltpu.sync_copy(w1_hbm.at[pl.ds(det0, DETS_PER_TILE), pl.ds(col0, CHUNK)],
                        w1_loc)

        def pvec_body(i, _):
            col = i * 16
            accs = [jnp.zeros((16,), jnp.float32) for _ in range(B)]
            for d in range(DETS_PER_TILE):
                k_vec = k_loc[d, pl.ds(col, 16)]
                w0_vec = w0_loc[d, pl.ds(col, 16)]
                w1_vec = w1_loc[d, pl.ds(col, 16)]
                for b in range(B):
                    row = jnp.full((16,), d * B + b, jnp.int32)
                    s0 = plsc.load_gather(s_loc, [row, k_vec])
                    s1 = plsc.load_gather(s_loc, [row, k_vec + 1])
                    accs[b] = accs[b] + w0_vec * s0 + w1_vec * s1
            for b in range(B):
                acc_loc[b, col // 256, pl.ds(col % 256, 16)] = accs[b]
            return 0

        lax.fori_loop(0, PVECS, pvec_body, 0)

        pltpu.sync_copy(
            acc_loc,
            part_hbm.at[sid, :, cid, pl.ds(c * (CHUNK // 256), CHUNK // 256), :])
        return 0

    lax.fori_loop(0, N_CHUNKS, chunk_body, 0)


def _reduce_body(part_ref, out_ref):
    out_ref[...] = jnp.sum(part_ref[...], axis=0)


@jax.jit
def kernel(sino, alpha, apod, k0, valid):
    sino_r = sino.reshape(B, N_DET, N_T)

    s_t = pl.pallas_call(
        _normalize_body,
        out_shape=jax.ShapeDtypeStruct((N_DET, B, N_T), jnp.float32),
    )(sino_r)
    s_flat = s_t.reshape(N_DET * B, N_T)

    PB = 2048
    alpha_r = alpha.reshape(P, N_DET)
    valid_u8 = valid.reshape(P, N_DET).astype(jnp.uint8)
    k0_r = k0.reshape(P, N_DET)
    apod_r = apod.reshape(1, N_DET)
    w0, w1, k0t = pl.pallas_call(
        _lut_body,
        grid=(P // PB,),
        in_specs=[
            pl.BlockSpec((PB, N_DET), lambda p: (p, 0)),
            pl.BlockSpec((PB, N_DET), lambda p: (p, 0)),
            pl.BlockSpec((PB, N_DET), lambda p: (p, 0)),
            pl.BlockSpec((1, N_DET), lambda p: (0, 0)),
        ],
        out_specs=[
            pl.BlockSpec((N_DET, PB), lambda p: (0, p)),
            pl.BlockSpec((N_DET, PB), lambda p: (0, p)),
            pl.BlockSpec((N_DET, PB), lambda p: (0, p)),
        ],
        out_shape=[
            jax.ShapeDtypeStruct((N_DET, P), jnp.float32),
            jax.ShapeDtypeStruct((N_DET, P), jnp.float32),
            jax.ShapeDtypeStruct((N_DET, P), jnp.int32),
        ],
    )(alpha_r, valid_u8, k0_r, apod_r)

    mesh = plsc.VectorSubcoreMesh(core_axis_name="c", subcore_axis_name="s",
                                  num_cores=NC, num_subcores=NS)
    part = pl.kernel(
        _sc_body,
        out_type=jax.ShapeDtypeStruct((NS, B, NC, P_HALF // 256, 256),
                                      jnp.float32),
        mesh=mesh,
        compiler_params=pltpu.CompilerParams(needs_layout_passes=False),
        scratch_types=[
            pltpu.VMEM((DETS_PER_TILE * B, N_T), jnp.float32),
            pltpu.VMEM((DETS_PER_TILE, CHUNK), jnp.int32),
            pltpu.VMEM((DETS_PER_TILE, CHUNK), jnp.float32),
            pltpu.VMEM((DETS_PER_TILE, CHUNK), jnp.float32),
            pltpu.VMEM((B, CHUNK // 256, 256), jnp.float32),
        ],
    )(s_flat, k0t, w0, w1)

    part_r = part.reshape(NS, B, P)
    PRB = 2048
    out = pl.pallas_call(
        _reduce_body,
        grid=(P // PRB,),
        in_specs=[pl.BlockSpec((NS, B, PRB), lambda p: (0, 0, p))],
        out_specs=pl.BlockSpec((B, PRB), lambda p: (0, p)),
        out_shape=jax.ShapeDtypeStruct((B, P), jnp.float32),
    )(part_r)

    return out.reshape(B, 1, NY, NX)


# speedup vs baseline: 825.8021x; 1.3279x over previous
import jax
import jax.numpy as jnp
from jax import lax
from jax.experimental import pallas as pl
from jax.experimental.pallas import tpu as pltpu
from jax.experimental.pallas import tpu_sc as plsc

B = 4
N_DET = 128
N_T = 2048
NY = 256
NX = 256
P = NY * NX
NC = 2
NS = 16
DETS_PER_TILE = N_DET // NS
P_HALF = P // NC
CHUNK = 1024
N_CHUNKS = P_HALF // CHUNK
PVECS = CHUNK // 16


def _pack_pair(lo_f32, hi_f32):
    lo = lax.bitcast_convert_type(lo_f32.astype(jnp.bfloat16), jnp.uint16)
    hi = lax.bitcast_convert_type(hi_f32.astype(jnp.bfloat16), jnp.uint16)
    pair = lo.astype(jnp.uint32) | (hi.astype(jnp.uint32) << 16)
    return lax.bitcast_convert_type(pair, jnp.int32)


def _normalize_body(sino_ref, out_ref):
    x = sino_ref[...]
    eps = jnp.finfo(jnp.float32).eps
    m = jnp.mean(x, axis=(1, 2), keepdims=True)
    c = x - m
    var = jnp.mean(c * c, axis=(1, 2), keepdims=True)
    s = c / jnp.sqrt(var + eps)
    s_next = jnp.concatenate(
        [s[..., 1:], jnp.zeros((B, N_DET, 1), jnp.float32)], axis=-1)
    pair = _pack_pair(s, s_next)
    out_ref[...] = jnp.transpose(pair, (1, 0, 2))


def _lut_body(alpha_ref, valid_ref, k0_ref, apod_ref, wp_ref, k0t_ref):
    a = alpha_ref[...]
    v = valid_ref[...].astype(jnp.float32)
    ap = apod_ref[...]
    norm = jnp.maximum(jnp.sum(ap), jnp.finfo(jnp.float32).tiny)
    base = v * (ap / norm)
    w1 = base * a
    w0 = base - w1
    wp_ref[...] = _pack_pair(w0, w1).T
    k0t_ref[...] = k0_ref[...].T


def _sc_body(s_hbm, k0_hbm, wp_hbm, part_hbm,
             s_loc, k_a, w_a, k_b, w_b, acc_loc, sem_a, sem_b):
    cid = lax.axis_index("c")
    sid = lax.axis_index("s")
    det0 = sid * DETS_PER_TILE

    pltpu.sync_copy(s_hbm.at[pl.ds(det0 * B, DETS_PER_TILE * B), :], s_loc)

    def _slices(c):
        col0 = cid * P_HALF + c * CHUNK
        ksl = k0_hbm.at[pl.ds(det0, DETS_PER_TILE), pl.ds(col0, CHUNK)]
        wsl = wp_hbm.at[pl.ds(det0, DETS_PER_TILE), pl.ds(col0, CHUNK)]
        return ksl, wsl

    def issue(c, kbuf, wbuf, sem):
        ksl, wsl = _slices(c)
        pltpu.async_copy(ksl, kbuf, sem)
        pltpu.async_copy(wsl, wbuf, sem)

    def drain(c, kbuf, wbuf, sem):
        ksl, wsl = _slices(c)
        pltpu.make_async_copy(ksl, kbuf, sem).wait()
        pltpu.make_async_copy(wsl, wbuf, sem).wait()

    def compute(c, kbuf, wbuf):
        def pvec_body(i, _):
            col = i * 16
            accs = [jnp.zeros((16,), jnp.float32) for _ in range(B)]
            for d in range(DETS_PER_TILE):
                k_vec = kbuf[d, pl.ds(col, 16)]
                w_pair = plsc.bitcast(wbuf[d, pl.ds(col, 16)], jnp.bfloat16)
                w0_vec, w1_vec = plsc.unpack(
                    w_pair, format=plsc.PackFormat.INTERLEAVED)
                for b in range(B):
                    row = jnp.full((16,), d * B + b, jnp.int32)
                    pair = plsc.bitcast(
                        plsc.load_gather(s_loc, [row, k_vec]), jnp.bfloat16)
                    s0, s1 = plsc.unpack(
                        pair, format=plsc.PackFormat.INTERLEAVED)
                    accs[b] = accs[b] + w0_vec * s0 + w1_vec * s1
            for b in range(B):
                acc_loc[b, col // 256, pl.ds(col % 256, 16)] = accs[b]
            return 0

        lax.fori_loop(0, PVECS, pvec_body, 0)
        pltpu.sync_copy(
            acc_loc,
            part_hbm.at[sid, :, cid, pl.ds(c * (CHUNK // 256), CHUNK // 256), :])

    issue(0, k_a, w_a, sem_a)

    def pair_body(h, _):
        g0 = 2 * h
        issue(g0 + 1, k_b, w_b, sem_b)
        drain(g0, k_a, w_a, sem_a)
        compute(g0, k_a, w_a)

        @pl.when(h < N_CHUNKS // 2 - 1)
        def _():
            issue(g0 + 2, k_a, w_a, sem_a)

        drain(g0 + 1, k_b, w_b, sem_b)
        compute(g0 + 1, k_b, w_b)
        return 0

    lax.fori_loop(0, N_CHUNKS // 2, pair_body, 0)


def _reduce_body(part_ref, out_ref):
    out_ref[...] = jnp.sum(part_ref[...], axis=0)


@jax.jit
def kernel(sino, alpha, apod, k0, valid):
    sino_r = sino.reshape(B, N_DET, N_T)

    s_t = pl.pallas_call(
        _normalize_body,
        out_shape=jax.ShapeDtypeStruct((N_DET, B, N_T), jnp.int32),
    )(sino_r)
    s_flat = s_t.reshape(N_DET * B, N_T)

    PB = 2048
    alpha_r = alpha.reshape(P, N_DET)
    valid_u8 = valid.reshape(P, N_DET).astype(jnp.uint8)
    k0_r = k0.reshape(P, N_DET)
    apod_r = apod.reshape(1, N_DET)
    wp, k0t = pl.pallas_call(
        _lut_body,
        grid=(P // PB,),
        in_specs=[
            pl.BlockSpec((PB, N_DET), lambda p: (p, 0)),
            pl.BlockSpec((PB, N_DET), lambda p: (p, 0)),
            pl.BlockSpec((PB, N_DET), lambda p: (p, 0)),
            pl.BlockSpec((1, N_DET), lambda p: (0, 0)),
        ],
        out_specs=[
            pl.BlockSpec((N_DET, PB), lambda p: (0, p)),
            pl.BlockSpec((N_DET, PB), lambda p: (0, p)),
        ],
        out_shape=[
            jax.ShapeDtypeStruct((N_DET, P), jnp.int32),
            jax.ShapeDtypeStruct((N_DET, P), jnp.int32),
        ],
    )(alpha_r, valid_u8, k0_r, apod_r)

    mesh = plsc.VectorSubcoreMesh(core_axis_name="c", subcore_axis_name="s",
                                  num_cores=NC, num_subcores=NS)
    part = pl.kernel(
        _sc_body,
        out_type=jax.ShapeDtypeStruct((NS, B, NC, P_HALF // 256, 256),
                                      jnp.float32),
        mesh=mesh,
        compiler_params=pltpu.CompilerParams(needs_layout_passes=False),
        scratch_types=[
            pltpu.VMEM((DETS_PER_TILE * B, N_T), jnp.int32),
            pltpu.VMEM((DETS_PER_TILE, CHUNK), jnp.int32),
            pltpu.VMEM((DETS_PER_TILE, CHUNK), jnp.int32),
            pltpu.VMEM((DETS_PER_TILE, CHUNK), jnp.int32),
            pltpu.VMEM((DETS_PER_TILE, CHUNK), jnp.int32),
            pltpu.VMEM((B, CHUNK // 256, 256), jnp.float32),
            pltpu.SemaphoreType.DMA,
            pltpu.SemaphoreType.DMA,
        ],
    )(s_flat, k0t, wp)

    part_r = part.reshape(NS, B, P)
    PRB = 2048
    out = pl.pallas_call(
        _reduce_body,
        grid=(P // PRB,),
        in_specs=[pl.BlockSpec((NS, B, PRB), lambda p: (0, 0, p))],
        out_specs=pl.BlockSpec((B, PRB), lambda p: (0, p)),
        out_shape=jax.ShapeDtypeStruct((B, P), jnp.float32),
    )(part_r)

    return out.reshape(B, 1, NY, NX)


# speedup vs baseline: 834.4531x; 1.0105x over previous
import jax
import jax.numpy as jnp
from jax import lax
from jax.experimental import pallas as pl
from jax.experimental.pallas import tpu as pltpu
from jax.experimental.pallas import tpu_sc as plsc

B = 4
N_DET = 128
N_T = 2048
NY = 256
NX = 256
P = NY * NX
NC = 2
NS = 16
DETS_PER_TILE = N_DET // NS
P_HALF = P // NC
CHUNK = 1024
N_CHUNKS = P_HALF // CHUNK
PVECS = CHUNK // 16


def _pack_pair(lo_f32, hi_f32):
    lo = lax.bitcast_convert_type(lo_f32.astype(jnp.bfloat16), jnp.uint16)
    hi = lax.bitcast_convert_type(hi_f32.astype(jnp.bfloat16), jnp.uint16)
    pair = lo.astype(jnp.uint32) | (hi.astype(jnp.uint32) << 16)
    return lax.bitcast_convert_type(pair, jnp.int32)


def _normalize_body(sino_ref, out_ref):
    x = sino_ref[...]
    eps = jnp.finfo(jnp.float32).eps
    m = jnp.mean(x, axis=(1, 2), keepdims=True)
    c = x - m
    var = jnp.mean(c * c, axis=(1, 2), keepdims=True)
    s = c / jnp.sqrt(var + eps)
    s_next = jnp.concatenate(
        [s[..., 1:], jnp.zeros((B, N_DET, 1), jnp.float32)], axis=-1)
    pair = _pack_pair(s, s_next)
    out_ref[...] = jnp.transpose(pair, (1, 0, 2))


def _lut_body(alpha_ref, valid_ref, k0_ref, apod_ref, wp_ref, k0t_ref):
    a = alpha_ref[...]
    v = valid_ref[...].astype(jnp.float32)
    ap = apod_ref[...]
    norm = jnp.maximum(jnp.sum(ap), jnp.finfo(jnp.float32).tiny)
    base = v * (ap / norm)
    w1 = base * a
    w0 = base - w1
    wp_ref[...] = _pack_pair(w0, w1).T
    k0t_ref[...] = k0_ref[...].T


def _sc_body(s_hbm, k0_hbm, wp_hbm, part_hbm,
             s_loc, k_a, w_a, k_b, w_b, acc_loc, sem_a, sem_b):
    cid = lax.axis_index("c")
    sid = lax.axis_index("s")
    det0 = sid * DETS_PER_TILE

    pltpu.sync_copy(s_hbm.at[pl.ds(det0 * B, DETS_PER_TILE * B), :], s_loc)

    def _slices(c):
        col0 = cid * P_HALF + c * CHUNK
        ksl = k0_hbm.at[pl.ds(det0, DETS_PER_TILE), pl.ds(col0, CHUNK)]
        wsl = wp_hbm.at[pl.ds(det0, DETS_PER_TILE), pl.ds(col0, CHUNK)]
        return ksl, wsl

    def issue(c, kbuf, wbuf, sem):
        ksl, wsl = _slices(c)
        pltpu.async_copy(ksl, kbuf, sem)
        pltpu.async_copy(wsl, wbuf, sem)

    def drain(c, kbuf, wbuf, sem):
        ksl, wsl = _slices(c)
        pltpu.make_async_copy(ksl, kbuf, sem).wait()
        pltpu.make_async_copy(wsl, wbuf, sem).wait()

    def compute(c, kbuf, wbuf):
        @plsc.parallel_loop(0, PVECS, unroll=2)
        def pvec_body(i):
            col = i * 16
            accs = [[jnp.zeros((16,), jnp.float32) for _ in range(2)]
                    for _ in range(B)]
            for d in range(DETS_PER_TILE):
                k_vec = kbuf[d, pl.ds(col, 16)]
                w_pair = plsc.bitcast(wbuf[d, pl.ds(col, 16)], jnp.bfloat16)
                w0_vec, w1_vec = plsc.unpack(
                    w_pair, format=plsc.PackFormat.INTERLEAVED)
                for b in range(B):
                    row = jnp.full((16,), d * B + b, jnp.int32)
                    pair = plsc.bitcast(
                        plsc.load_gather(s_loc, [row, k_vec]), jnp.bfloat16)
                    s0, s1 = plsc.unpack(
                        pair, format=plsc.PackFormat.INTERLEAVED)
                    accs[b][d % 2] = accs[b][d % 2] + w0_vec * s0 + w1_vec * s1
            for b in range(B):
                acc_loc[b, col // 256, pl.ds(col % 256, 16)] = \
                    accs[b][0] + accs[b][1]
        pltpu.sync_copy(
            acc_loc,
            part_hbm.at[sid, :, cid, pl.ds(c * (CHUNK // 256), CHUNK // 256), :])

    issue(0, k_a, w_a, sem_a)

    def pair_body(h, _):
        g0 = 2 * h
        issue(g0 + 1, k_b, w_b, sem_b)
        drain(g0, k_a, w_a, sem_a)
        compute(g0, k_a, w_a)

        @pl.when(h < N_CHUNKS // 2 - 1)
        def _():
            issue(g0 + 2, k_a, w_a, sem_a)

        drain(g0 + 1, k_b, w_b, sem_b)
        compute(g0 + 1, k_b, w_b)
        return 0

    lax.fori_loop(0, N_CHUNKS // 2, pair_body, 0)


def _reduce_body(part_ref, out_ref):
    out_ref[...] = jnp.sum(part_ref[...], axis=0)


@jax.jit
def kernel(sino, alpha, apod, k0, valid):
    sino_r = sino.reshape(B, N_DET, N_T)

    s_t = pl.pallas_call(
        _normalize_body,
        out_shape=jax.ShapeDtypeStruct((N_DET, B, N_T), jnp.int32),
    )(sino_r)
    s_flat = s_t.reshape(N_DET * B, N_T)

    PB = 2048
    alpha_r = alpha.reshape(P, N_DET)
    valid_u8 = valid.reshape(P, N_DET).astype(jnp.uint8)
    k0_r = k0.reshape(P, N_DET)
    apod_r = apod.reshape(1, N_DET)
    wp, k0t = pl.pallas_call(
        _lut_body,
        grid=(P // PB,),
        in_specs=[
            pl.BlockSpec((PB, N_DET), lambda p: (p, 0)),
            pl.BlockSpec((PB, N_DET), lambda p: (p, 0)),
            pl.BlockSpec((PB, N_DET), lambda p: (p, 0)),
            pl.BlockSpec((1, N_DET), lambda p: (0, 0)),
        ],
        out_specs=[
            pl.BlockSpec((N_DET, PB), lambda p: (0, p)),
            pl.BlockSpec((N_DET, PB), lambda p: (0, p)),
        ],
        out_shape=[
            jax.ShapeDtypeStruct((N_DET, P), jnp.int32),
            jax.ShapeDtypeStruct((N_DET, P), jnp.int32),
        ],
    )(alpha_r, valid_u8, k0_r, apod_r)

    mesh = plsc.VectorSubcoreMesh(core_axis_name="c", subcore_axis_name="s",
                                  num_cores=NC, num_subcores=NS)
    part = pl.kernel(
        _sc_body,
        out_type=jax.ShapeDtypeStruct((NS, B, NC, P_HALF // 256, 256),
                                      jnp.float32),
        mesh=mesh,
        compiler_params=pltpu.CompilerParams(needs_layout_passes=False),
        scratch_types=[
            pltpu.VMEM((DETS_PER_TILE * B, N_T), jnp.int32),
            pltpu.VMEM((DETS_PER_TILE, CHUNK), jnp.int32),
            pltpu.VMEM((DETS_PER_TILE, CHUNK), jnp.int32),
            pltpu.VMEM((DETS_PER_TILE, CHUNK), jnp.int32),
            pltpu.VMEM((DETS_PER_TILE, CHUNK), jnp.int32),
            pltpu.VMEM((B, CHUNK // 256, 256), jnp.float32),
            pltpu.SemaphoreType.DMA,
            pltpu.SemaphoreType.DMA,
        ],
    )(s_flat, k0t, wp)

    part_r = part.reshape(NS, B, P)
    PRB = 2048
    out = pl.pallas_call(
        _reduce_body,
        grid=(P // PRB,),
        in_specs=[pl.BlockSpec((NS, B, PRB), lambda p: (0, 0, p))],
        out_specs=pl.BlockSpec((B, PRB), lambda p: (0, p)),
        out_shape=jax.ShapeDtypeStruct((B, P), jnp.float32),
    )(part_r)

    return out.reshape(B, 1, NY, NX)


# speedup vs baseline: 1088.7479x; 1.3047x over previous
import jax
import jax.numpy as jnp
from jax import lax
from jax.experimental import pallas as pl
from jax.experimental.pallas import tpu as pltpu
from jax.experimental.pallas import tpu_sc as plsc

B = 4
N_DET = 128
N_T = 2048
NY = 256
NX = 256
P = NY * NX
NC = 2
NS = 16
DETS_PER_TILE = N_DET // NS
P_HALF = P // NC
CHUNK = 1024
N_CHUNKS = P_HALF // CHUNK
PVECS = CHUNK // 16


def _pack_pair(lo_f32, hi_f32):
    lo = lax.bitcast_convert_type(lo_f32.astype(jnp.bfloat16), jnp.uint16)
    hi = lax.bitcast_convert_type(hi_f32.astype(jnp.bfloat16), jnp.uint16)
    pair = lo.astype(jnp.uint32) | (hi.astype(jnp.uint32) << 16)
    return lax.bitcast_convert_type(pair, jnp.int32)


def _normalize_body(sino_ref, out_ref):
    x = sino_ref[...]
    eps = jnp.finfo(jnp.float32).eps
    m = jnp.mean(x, axis=(1, 2), keepdims=True)
    c = x - m
    var = jnp.mean(c * c, axis=(1, 2), keepdims=True)
    s = c / jnp.sqrt(var + eps)
    s_next = jnp.concatenate(
        [s[..., 1:], jnp.zeros((B, N_DET, 1), jnp.float32)], axis=-1)
    pair = _pack_pair(s, s_next)
    out_ref[...] = jnp.transpose(pair, (1, 0, 2))


def _lut_body(alpha_ref, valid_ref, k0_ref, apod_ref, wp_ref, k0t_ref):
    a = alpha_ref[...]
    v = valid_ref[...].astype(jnp.float32)
    ap = apod_ref[...]
    norm = jnp.maximum(jnp.sum(ap), jnp.finfo(jnp.float32).tiny)
    base = v * (ap / norm)
    w1 = base * a
    w0 = base - w1
    wp_ref[...] = _pack_pair(w0, w1).T
    k0t_ref[...] = k0_ref[...].T


def _sc_body(s_hbm, k0_hbm, wp_hbm, part_hbm,
             s_loc, k_a, w_a, k_b, w_b, acc_loc, sem_a, sem_b):
    cid = lax.axis_index("c")
    sid = lax.axis_index("s")
    det0 = sid * DETS_PER_TILE

    pltpu.sync_copy(s_hbm.at[pl.ds(det0 * B, DETS_PER_TILE * B), :], s_loc)

    def _slices(c):
        col0 = cid * P_HALF + c * CHUNK
        ksl = k0_hbm.at[pl.ds(det0, DETS_PER_TILE), pl.ds(col0, CHUNK)]
        wsl = wp_hbm.at[pl.ds(det0, DETS_PER_TILE), pl.ds(col0, CHUNK)]
        return ksl, wsl

    def issue(c, kbuf, wbuf, sem):
        ksl, wsl = _slices(c)
        pltpu.async_copy(ksl, kbuf, sem)
        pltpu.async_copy(wsl, wbuf, sem)

    def drain(c, kbuf, wbuf, sem):
        ksl, wsl = _slices(c)
        pltpu.make_async_copy(ksl, kbuf, sem).wait()
        pltpu.make_async_copy(wsl, wbuf, sem).wait()

    def compute(c, kbuf, wbuf):
        @plsc.parallel_loop(0, PVECS, unroll=2)
        def pvec_body(i):
            col = i * 16
            accs = [[jnp.zeros((32,), jnp.bfloat16) for _ in range(2)]
                    for _ in range(B)]
            for d in range(DETS_PER_TILE):
                k_vec = kbuf[d, pl.ds(col, 16)]
                w_pair = plsc.bitcast(wbuf[d, pl.ds(col, 16)], jnp.bfloat16)
                for b in range(B):
                    row = jnp.full((16,), d * B + b, jnp.int32)
                    pair = plsc.bitcast(
                        plsc.load_gather(s_loc, [row, k_vec]), jnp.bfloat16)
                    accs[b][d % 2] = accs[b][d % 2] + w_pair * pair
            for b in range(B):
                t0, t1 = plsc.unpack(accs[b][0] + accs[b][1],
                                     format=plsc.PackFormat.INTERLEAVED)
                acc_loc[b, col // 256, pl.ds(col % 256, 16)] = t0 + t1
        pltpu.sync_copy(
            acc_loc,
            part_hbm.at[sid, :, cid, pl.ds(c * (CHUNK // 256), CHUNK // 256), :])

    issue(0, k_a, w_a, sem_a)

    def pair_body(h, _):
        g0 = 2 * h
        issue(g0 + 1, k_b, w_b, sem_b)
        drain(g0, k_a, w_a, sem_a)
        compute(g0, k_a, w_a)

        @pl.when(h < N_CHUNKS // 2 - 1)
        def _():
            issue(g0 + 2, k_a, w_a, sem_a)

        drain(g0 + 1, k_b, w_b, sem_b)
        compute(g0 + 1, k_b, w_b)
        return 0

    lax.fori_loop(0, N_CHUNKS // 2, pair_body, 0)


def _reduce_body(part_ref, out_ref):
    out_ref[...] = jnp.sum(part_ref[...], axis=0)


@jax.jit
def kernel(sino, alpha, apod, k0, valid):
    sino_r = sino.reshape(B, N_DET, N_T)

    s_t = pl.pallas_call(
        _normalize_body,
        out_shape=jax.ShapeDtypeStruct((N_DET, B, N_T), jnp.int32),
    )(sino_r)
    s_flat = s_t.reshape(N_DET * B, N_T)

    PB = 2048
    alpha_r = alpha.reshape(P, N_DET)
    valid_u8 = valid.reshape(P, N_DET).astype(jnp.uint8)
    k0_r = k0.reshape(P, N_DET)
    apod_r = apod.reshape(1, N_DET)
    wp, k0t = pl.pallas_call(
        _lut_body,
        grid=(P // PB,),
        in_specs=[
            pl.BlockSpec((PB, N_DET), lambda p: (p, 0)),
            pl.BlockSpec((PB, N_DET), lambda p: (p, 0)),
            pl.BlockSpec((PB, N_DET), lambda p: (p, 0)),
            pl.BlockSpec((1, N_DET), lambda p: (0, 0)),
        ],
        out_specs=[
            pl.BlockSpec((N_DET, PB), lambda p: (0, p)),
            pl.BlockSpec((N_DET, PB), lambda p: (0, p)),
        ],
        out_shape=[
            jax.ShapeDtypeStruct((N_DET, P), jnp.int32),
            jax.ShapeDtypeStruct((N_DET, P), jnp.int32),
        ],
    )(alpha_r, valid_u8, k0_r, apod_r)

    mesh = plsc.VectorSubcoreMesh(core_axis_name="c", subcore_axis_name="s",
                                  num_cores=NC, num_subcores=NS)
    part = pl.kernel(
        _sc_body,
        out_type=jax.ShapeDtypeStruct((NS, B, NC, P_HALF // 256, 256),
                                      jnp.float32),
        mesh=mesh,
        compiler_params=pltpu.CompilerParams(needs_layout_passes=False),
        scratch_types=[
            pltpu.VMEM((DETS_PER_TILE * B, N_T), jnp.int32),
            pltpu.VMEM((DETS_PER_TILE, CHUNK), jnp.int32),
            pltpu.VMEM((DETS_PER_TILE, CHUNK), jnp.int32),
            pltpu.VMEM((DETS_PER_TILE, CHUNK), jnp.int32),
            pltpu.VMEM((DETS_PER_TILE, CHUNK), jnp.int32),
            pltpu.VMEM((B, CHUNK // 256, 256), jnp.float32),
            pltpu.SemaphoreType.DMA,
            pltpu.SemaphoreType.DMA,
        ],
    )(s_flat, k0t, wp)

    part_r = part.reshape(NS, B, P)
    PRB = 2048
    out = pl.pallas_call(
        _reduce_body,
        grid=(P // PRB,),
        in_specs=[pl.BlockSpec((NS, B, PRB), lambda p: (0, 0, p))],
        out_specs=pl.BlockSpec((B, PRB), lambda p: (0, p)),
        out_shape=jax.ShapeDtypeStruct((B, P), jnp.float32),
    )(part_r)

    return out.reshape(B, 1, NY, NX)
